# baseline (speedup 1.0000x reference)
import functools

import jax
import jax.numpy as jnp
from jax import lax
from jax.experimental import pallas as pl
from jax.experimental.pallas import tpu as pltpu
from jax.experimental.pallas import tpu_sc as plsc

B = 16384
E = 16
_NC = 2
_NS = 16
_NW = _NC * _NS
_BPW = B // _NW
_CHUNK = 128
_NCHUNK = _BPW // _CHUNK


def _fm_body(uid_hbm, iid_hbm, uemb_hbm, iemb_hbm, ubias_hbm, ibias_hbm,
             bias_hbm, out_hbm,
             idx_u, idx_i, u_rows, i_rows, u_b, i_b, bias_v, out_v, sem):
    wid = lax.axis_index("s") * _NC + lax.axis_index("c")

    pltpu.sync_copy(uid_hbm.at[pl.ds(wid * _NCHUNK, _NCHUNK)], idx_u)
    pltpu.sync_copy(iid_hbm.at[pl.ds(wid * _NCHUNK, _NCHUNK)], idx_i)
    pltpu.sync_copy(bias_hbm, bias_v)

    copies = []
    for c in range(_NCHUNK):
        copies.append(pltpu.async_copy(uemb_hbm.at[idx_u.at[c]], u_rows.at[c], sem))
        copies.append(pltpu.async_copy(iemb_hbm.at[idx_i.at[c]], i_rows.at[c], sem))
        copies.append(pltpu.async_copy(ubias_hbm.at[idx_u.at[c]], u_b.at[c], sem))
        copies.append(pltpu.async_copy(ibias_hbm.at[idx_i.at[c]], i_b.at[c], sem))
    for cp in copies:
        cp.wait()

    iota = lax.iota(jnp.int32, 16)
    ones = jnp.ones((16,), jnp.int32)
    bias0 = bias_v[...]

    def block(b, carry):
        c_vec = ones * (b >> 3)
        rows = (b & 7) * 16 + iota
        acc = (bias0
               + plsc.load_gather(u_b, [c_vec, rows])
               + plsc.load_gather(i_b, [c_vec, rows]))
        for e in range(E):
            elem = (iota + e) & 15
            uu = plsc.load_gather(u_rows, [c_vec, rows, elem])
            ii = plsc.load_gather(i_rows, [c_vec, rows, elem])
            acc = acc + uu * ii
        plsc.store_scatter(out_v, [b * 16 + iota], acc)
        return carry

    lax.fori_loop(0, _BPW // 16, block, 0)

    pltpu.sync_copy(out_v, out_hbm.at[pl.ds(wid * _BPW, _BPW)])


def kernel(u_ids, i_ids, user_emb, item_emb, user_bias, item_bias, bias):
    uid2 = u_ids.reshape(B // _CHUNK, _CHUNK)
    iid2 = i_ids.reshape(B // _CHUNK, _CHUNK)
    ub_flat = user_bias.reshape(-1)
    ib_flat = item_bias.reshape(-1)
    bias16 = jnp.broadcast_to(bias, (16,))

    mesh = plsc.VectorSubcoreMesh(core_axis_name="c", subcore_axis_name="s")
    fm = functools.partial(
        pl.kernel,
        mesh=mesh,
        compiler_params=pltpu.CompilerParams(
            needs_layout_passes=False, use_tc_tiling_on_sc=False),
        out_type=jax.ShapeDtypeStruct((B,), jnp.float32),
        scratch_types=[
            pltpu.VMEM((_NCHUNK, _CHUNK), jnp.int32),
            pltpu.VMEM((_NCHUNK, _CHUNK), jnp.int32),
            pltpu.VMEM((_NCHUNK, _CHUNK, E), jnp.float32),
            pltpu.VMEM((_NCHUNK, _CHUNK, E), jnp.float32),
            pltpu.VMEM((_NCHUNK, _CHUNK), jnp.float32),
            pltpu.VMEM((_NCHUNK, _CHUNK), jnp.float32),
            pltpu.VMEM((16,), jnp.float32),
            pltpu.VMEM((_BPW,), jnp.float32),
            pltpu.SemaphoreType.DMA,
        ],
    )(_fm_body)
    return fm(uid2, iid2, user_emb, item_emb, ub_flat, ib_flat, bias16)
